# baseline (speedup 1.0000x reference)
import jax
import jax.numpy as jnp
from jax.experimental import pallas as pl
from jax.experimental.pallas import tpu as pltpu
from jax.experimental.pallas import tpu_sc as plsc

_S, _V = 2048, 32000


def _copy_row(src_hbm, out_hbm, buf):
    pltpu.sync_copy(src_hbm.at[0, pl.ds(_S - 8, 8), :], buf)
    pltpu.sync_copy(buf.at[7], out_hbm.at[0, :])


def kernel(logits):
    k = pl.kernel(
        _copy_row,
        out_type=jax.ShapeDtypeStruct((1, _V), jnp.float32),
        mesh=plsc.ScalarSubcoreMesh(axis_name="c", num_cores=1),
        scratch_types=[pltpu.VMEM_SHARED((8, _V), jnp.float32)],
    )
    return k(logits)
